# baseline (speedup 1.0000x reference)
import functools

import jax
import jax.numpy as jnp
from jax import lax
from jax.experimental import pallas as pl
from jax.experimental.pallas import tpu as pltpu
from jax.experimental.pallas import tpu_sc as plsc

N_NODES = 100000
BATCH = 8192
SRC_DIM = 128
DST_DIM = 128

_SC_INFO = plsc.get_sparse_core_info()
_NC = _SC_INFO.num_cores
_NS = _SC_INFO.num_subcores
_NW = _NC * _NS

_HALF = BATCH // 2


def _make_sc_gather(chunks):
    mesh = plsc.VectorSubcoreMesh(core_axis_name="c", subcore_axis_name="s")
    bpw = [length // _NW for _, length in chunks]

    @functools.partial(
        pl.kernel,
        mesh=mesh,
        out_type=[jax.ShapeDtypeStruct((length, SRC_DIM), jnp.float32)
                  for _, length in chunks],
        scratch_types=(
            [pltpu.VMEM((b,), jnp.int32) for b in bpw]
            + [pltpu.VMEM((b, SRC_DIM), jnp.float32) for b in bpw]
            + [pltpu.SemaphoreType.DMA] * (3 * len(chunks))
        ),
    )
    def gather(table_hbm, *refs):
        n = len(chunks)
        idx_hbms = refs[:n]
        outs = refs[n:2 * n]
        idx_vs = refs[2 * n:3 * n]
        rows_vs = refs[3 * n:4 * n]
        sems = refs[4 * n:]
        wid = lax.axis_index("s") * _NC + lax.axis_index("c")
        idx_cps = []
        for k, (off, _) in enumerate(chunks):
            base = off + wid * bpw[k]
            idx_cps.append(pltpu.async_copy(
                idx_hbms[k].at[pl.ds(base, bpw[k])], idx_vs[k], sems[3 * k]))
        g_cps = []
        for k in range(n):
            idx_cps[k].wait()
            g_cps.append(pltpu.async_copy(
                table_hbm.at[idx_vs[k]], rows_vs[k], sems[3 * k + 1]))
        w_cps = []
        for k in range(n):
            g_cps[k].wait()
            w_cps.append(pltpu.async_copy(
                rows_vs[k], outs[k].at[pl.ds(wid * bpw[k], bpw[k])],
                sems[3 * k + 2]))
        for cp in w_cps:
            cp.wait()

    return gather


_sc_gather_a = _make_sc_gather([(0, _HALF)])
_sc_gather_b = _make_sc_gather([(_HALF, _HALF), (0, BATCH)])

_BLK = 256
_Q = _HALF // 2


def _tc_body1(d0, d1, src_ref, p_ref):
    p_ref[...] = (jnp.dot(d0[...], src_ref[:_Q, :],
                          preferred_element_type=jnp.float32)
                  + jnp.dot(d1[...], src_ref[_Q:, :],
                            preferred_element_type=jnp.float32))


def _tc_body2(d0, d1, src_ref, p_ref, dst_ref, w_ref, out_ref):
    agg = (p_ref[...]
           + jnp.dot(d0[...], src_ref[:_Q, :],
                     preferred_element_type=jnp.float32)
           + jnp.dot(d1[...], src_ref[_Q:, :],
                     preferred_element_type=jnp.float32))
    x = (jnp.dot(agg, w_ref[:SRC_DIM, :], preferred_element_type=jnp.float32)
         + jnp.dot(dst_ref[...], w_ref[SRC_DIM:, :],
                   preferred_element_type=jnp.float32))
    out_ref[...] = jnp.maximum(x, 0.0)


def kernel(dstsrc_features, dstsrc2src, dstsrc2dst, dif_mat, w):
    (src0,) = _sc_gather_a(dstsrc_features, dstsrc2src)
    src1, dst_f = _sc_gather_b(dstsrc_features, dstsrc2src, dstsrc2dst)
    nblk = BATCH // _BLK
    partial = pl.pallas_call(
        _tc_body1,
        grid=(nblk,),
        in_specs=[
            pl.BlockSpec((_BLK, _Q), lambda i: (i, 0)),
            pl.BlockSpec((_BLK, _Q), lambda i: (i, 1)),
            pl.BlockSpec((_HALF, SRC_DIM), lambda i: (0, 0)),
        ],
        out_specs=pl.BlockSpec((_BLK, DST_DIM), lambda i: (i, 0)),
        out_shape=jax.ShapeDtypeStruct((BATCH, DST_DIM), jnp.float32),
    )(dif_mat, dif_mat, src0)
    out = pl.pallas_call(
        _tc_body2,
        grid=(nblk,),
        in_specs=[
            pl.BlockSpec((_BLK, _Q), lambda i: (i, 2)),
            pl.BlockSpec((_BLK, _Q), lambda i: (i, 3)),
            pl.BlockSpec((_HALF, SRC_DIM), lambda i: (0, 0)),
            pl.BlockSpec((_BLK, DST_DIM), lambda i: (i, 0)),
            pl.BlockSpec((_BLK, SRC_DIM), lambda i: (i, 0)),
            pl.BlockSpec((2 * SRC_DIM, DST_DIM), lambda i: (0, 0)),
        ],
        out_specs=pl.BlockSpec((_BLK, DST_DIM), lambda i: (i, 0)),
        out_shape=jax.ShapeDtypeStruct((BATCH, DST_DIM), jnp.float32),
    )(dif_mat, dif_mat, src1, partial, dst_f, w)
    return out


# speedup vs baseline: 1.1438x; 1.1438x over previous
import functools

import jax
import jax.numpy as jnp
from jax import lax
from jax.experimental import pallas as pl
from jax.experimental.pallas import tpu as pltpu
from jax.experimental.pallas import tpu_sc as plsc

N_NODES = 100000
BATCH = 8192
SRC_DIM = 128
DST_DIM = 128

_SC_INFO = plsc.get_sparse_core_info()
_NC = _SC_INFO.num_cores
_NS = _SC_INFO.num_subcores
_NW = _NC * _NS

_HALF = BATCH // 2


def _make_sc_gather(chunks):
    mesh = plsc.VectorSubcoreMesh(core_axis_name="c", subcore_axis_name="s")
    bpw = [length // _NW for _, length in chunks]

    @functools.partial(
        pl.kernel,
        mesh=mesh,
        out_type=[jax.ShapeDtypeStruct((length, SRC_DIM), jnp.float32)
                  for _, length in chunks],
        scratch_types=(
            [pltpu.VMEM((b,), jnp.int32) for b in bpw]
            + [pltpu.VMEM((b, SRC_DIM), jnp.float32) for b in bpw]
            + [pltpu.SemaphoreType.DMA] * (3 * len(chunks))
        ),
    )
    def gather(table_hbm, *refs):
        n = len(chunks)
        idx_hbms = refs[:n]
        outs = refs[n:2 * n]
        idx_vs = refs[2 * n:3 * n]
        rows_vs = refs[3 * n:4 * n]
        sems = refs[4 * n:]
        wid = lax.axis_index("s") * _NC + lax.axis_index("c")
        idx_cps = []
        for k, (off, _) in enumerate(chunks):
            base = off + wid * bpw[k]
            idx_cps.append(pltpu.async_copy(
                idx_hbms[k].at[pl.ds(base, bpw[k])], idx_vs[k], sems[3 * k]))
        g_cps = []
        for k in range(n):
            idx_cps[k].wait()
            g_cps.append(pltpu.async_copy(
                table_hbm.at[idx_vs[k]], rows_vs[k], sems[3 * k + 1]))
        w_cps = []
        for k in range(n):
            g_cps[k].wait()
            w_cps.append(pltpu.async_copy(
                rows_vs[k], outs[k].at[pl.ds(wid * bpw[k], bpw[k])],
                sems[3 * k + 2]))
        for cp in w_cps:
            cp.wait()

    return gather


_sc_gather_a = _make_sc_gather([(0, _HALF)])
_sc_gather_b = _make_sc_gather([(_HALF, _HALF), (0, BATCH)])

_BLK = 512
_Q = _HALF // 2


def _tc_body1(d0, d1, src_ref, p_ref):
    p_ref[...] = (jnp.dot(d0[...], src_ref[:_Q, :],
                          preferred_element_type=jnp.float32)
                  + jnp.dot(d1[...], src_ref[_Q:, :],
                            preferred_element_type=jnp.float32))


def _tc_body2(d0, d1, src_ref, p_ref, dst_ref, w_ref, out_ref):
    agg = (p_ref[...]
           + jnp.dot(d0[...], src_ref[:_Q, :],
                     preferred_element_type=jnp.float32)
           + jnp.dot(d1[...], src_ref[_Q:, :],
                     preferred_element_type=jnp.float32))
    x = (jnp.dot(agg, w_ref[:SRC_DIM, :], preferred_element_type=jnp.float32)
         + jnp.dot(dst_ref[...], w_ref[SRC_DIM:, :],
                   preferred_element_type=jnp.float32))
    out_ref[...] = jnp.maximum(x, 0.0)


def kernel(dstsrc_features, dstsrc2src, dstsrc2dst, dif_mat, w):
    (src0,) = _sc_gather_a(dstsrc_features, dstsrc2src)
    src1, dst_f = _sc_gather_b(dstsrc_features, dstsrc2src, dstsrc2dst)
    nblk = BATCH // _BLK
    partial = pl.pallas_call(
        _tc_body1,
        grid=(nblk,),
        in_specs=[
            pl.BlockSpec((_BLK, _Q), lambda i: (i, 0)),
            pl.BlockSpec((_BLK, _Q), lambda i: (i, 1)),
            pl.BlockSpec((_HALF, SRC_DIM), lambda i: (0, 0)),
        ],
        out_specs=pl.BlockSpec((_BLK, DST_DIM), lambda i: (i, 0)),
        out_shape=jax.ShapeDtypeStruct((BATCH, DST_DIM), jnp.float32),
    )(dif_mat, dif_mat, src0)
    out = pl.pallas_call(
        _tc_body2,
        grid=(nblk,),
        in_specs=[
            pl.BlockSpec((_BLK, _Q), lambda i: (i, 2)),
            pl.BlockSpec((_BLK, _Q), lambda i: (i, 3)),
            pl.BlockSpec((_HALF, SRC_DIM), lambda i: (0, 0)),
            pl.BlockSpec((_BLK, DST_DIM), lambda i: (i, 0)),
            pl.BlockSpec((_BLK, SRC_DIM), lambda i: (i, 0)),
            pl.BlockSpec((2 * SRC_DIM, DST_DIM), lambda i: (0, 0)),
        ],
        out_specs=pl.BlockSpec((_BLK, DST_DIM), lambda i: (i, 0)),
        out_shape=jax.ShapeDtypeStruct((BATCH, DST_DIM), jnp.float32),
    )(dif_mat, dif_mat, src1, partial, dst_f, w)
    return out


# speedup vs baseline: 1.1907x; 1.0409x over previous
import functools

import jax
import jax.numpy as jnp
from jax import lax
from jax.experimental import pallas as pl
from jax.experimental.pallas import tpu as pltpu
from jax.experimental.pallas import tpu_sc as plsc

N_NODES = 100000
BATCH = 8192
SRC_DIM = 128
DST_DIM = 128

_SC_INFO = plsc.get_sparse_core_info()
_NC = _SC_INFO.num_cores
_NS = _SC_INFO.num_subcores
_NW = _NC * _NS
_BPW = BATCH // _NW


def _make_sc_gather2():
    mesh = plsc.VectorSubcoreMesh(core_axis_name="c", subcore_axis_name="s")

    @functools.partial(
        pl.kernel,
        mesh=mesh,
        out_type=[
            jax.ShapeDtypeStruct((BATCH, SRC_DIM), jnp.float32),
            jax.ShapeDtypeStruct((BATCH, SRC_DIM), jnp.float32),
        ],
        scratch_types=[
            pltpu.VMEM((_BPW,), jnp.int32),
            pltpu.VMEM((_BPW,), jnp.int32),
            pltpu.VMEM((_BPW, SRC_DIM), jnp.float32),
            pltpu.VMEM((_BPW, SRC_DIM), jnp.float32),
            pltpu.SemaphoreType.DMA,
            pltpu.SemaphoreType.DMA,
            pltpu.SemaphoreType.DMA,
            pltpu.SemaphoreType.DMA,
            pltpu.SemaphoreType.DMA,
            pltpu.SemaphoreType.DMA,
        ],
    )
    def gather2(table_hbm, src_idx_hbm, dst_idx_hbm, src_out, dst_out,
                sidx_v, didx_v, srows_v, drows_v,
                sem_i1, sem_i2, sem_s, sem_d, sem_ws, sem_wd):
        wid = lax.axis_index("s") * _NC + lax.axis_index("c")
        base = wid * _BPW
        ci = pltpu.async_copy(src_idx_hbm.at[pl.ds(base, _BPW)], sidx_v, sem_i1)
        cj = pltpu.async_copy(dst_idx_hbm.at[pl.ds(base, _BPW)], didx_v, sem_i2)
        ci.wait()
        cp_s = pltpu.async_copy(table_hbm.at[sidx_v], srows_v, sem_s)
        cj.wait()
        cp_d = pltpu.async_copy(table_hbm.at[didx_v], drows_v, sem_d)
        cp_s.wait()
        ws = pltpu.async_copy(srows_v, src_out.at[pl.ds(base, _BPW)], sem_ws)
        cp_d.wait()
        wd = pltpu.async_copy(drows_v, dst_out.at[pl.ds(base, _BPW)], sem_wd)
        ws.wait()
        wd.wait()

    return gather2


_sc_gather2 = _make_sc_gather2()

_BLK = 256
_QTR = BATCH // 4


def _tc_body(d0, d1, d2, d3, src_ref, dst_ref, w_ref, out_ref):
    agg = (jnp.dot(d0[...], src_ref[:_QTR, :],
                   preferred_element_type=jnp.float32)
           + jnp.dot(d1[...], src_ref[_QTR:2 * _QTR, :],
                     preferred_element_type=jnp.float32)
           + jnp.dot(d2[...], src_ref[2 * _QTR:3 * _QTR, :],
                     preferred_element_type=jnp.float32)
           + jnp.dot(d3[...], src_ref[3 * _QTR:, :],
                     preferred_element_type=jnp.float32))
    x = (jnp.dot(agg, w_ref[:SRC_DIM, :], preferred_element_type=jnp.float32)
         + jnp.dot(dst_ref[...], w_ref[SRC_DIM:, :],
                   preferred_element_type=jnp.float32))
    out_ref[...] = jnp.maximum(x, 0.0)


def kernel(dstsrc_features, dstsrc2src, dstsrc2dst, dif_mat, w):
    src_f, dst_f = _sc_gather2(dstsrc_features, dstsrc2src, dstsrc2dst)
    out = pl.pallas_call(
        _tc_body,
        grid=(BATCH // _BLK,),
        in_specs=[
            pl.BlockSpec((_BLK, _QTR), lambda i: (i, 0)),
            pl.BlockSpec((_BLK, _QTR), lambda i: (i, 1)),
            pl.BlockSpec((_BLK, _QTR), lambda i: (i, 2)),
            pl.BlockSpec((_BLK, _QTR), lambda i: (i, 3)),
            pl.BlockSpec((BATCH, SRC_DIM), lambda i: (0, 0)),
            pl.BlockSpec((_BLK, SRC_DIM), lambda i: (i, 0)),
            pl.BlockSpec((2 * SRC_DIM, DST_DIM), lambda i: (0, 0)),
        ],
        out_specs=pl.BlockSpec((_BLK, DST_DIM), lambda i: (i, 0)),
        out_shape=jax.ShapeDtypeStruct((BATCH, DST_DIM), jnp.float32),
    )(dif_mat, dif_mat, dif_mat, dif_mat, src_f, dst_f, w)
    return out
